# baseline (speedup 1.0000x reference)
import functools
import math

import jax
import jax.numpy as jnp
from jax import lax
from jax.experimental import pallas as pl
from jax.experimental.pallas import tpu as pltpu
from jax.experimental.pallas import tpu_sc as plsc

DIM = 64
SCALE = math.sqrt(DIM)

NC = 2
NS = 16
NW = NC * NS
LANES = 16
TB = 256


@functools.lru_cache(maxsize=None)
def _build_transpose(vocab: int):
    n_full = vocab // TB
    tail = vocab - n_full * TB
    iters = -(-n_full // NW)

    mesh = plsc.VectorSubcoreMesh(core_axis_name="c", subcore_axis_name="s")

    scratch = (
        [pltpu.VMEM((DIM, TB), jnp.float32) for _ in range(2)]
        + [pltpu.VMEM((TB, DIM), jnp.float32) for _ in range(2)]
        + [pltpu.SemaphoreType.DMA for _ in range(4)]
    )

    @functools.partial(
        pl.kernel,
        mesh=mesh,
        compiler_params=pltpu.CompilerParams(use_tc_tiling_on_sc=False, needs_layout_passes=False),
        out_type=jax.ShapeDtypeStruct((vocab, DIM), jnp.float32),
        scratch_types=scratch,
    )
    def tr_kernel(wt_hbm, wl_hbm, src0, src1, dst0, dst1, si0, si1, so0, so1):
        wid = lax.axis_index("s") * NC + lax.axis_index("c")
        srcs, dsts = (src0, src1), (dst0, dst1)
        isems, osems = (si0, si1), (so0, so1)
        iota = lax.iota(jnp.int32, LANES)

        if tail:
            @pl.when(wid == 0)
            def _():
                base = n_full * TB
                pltpu.sync_copy(wt_hbm.at[:, pl.ds(base, tail)],
                                src0.at[:, pl.ds(0, tail)])

                @plsc.parallel_loop(0, DIM, unroll=2)
                def d_loop(d):
                    for tg in range(tail // LANES):
                        v = src0[d, pl.ds(tg * LANES, LANES)] * SCALE
                        plsc.store_scatter(
                            dst0,
                            [iota + tg * LANES, jnp.full((LANES,), 0, jnp.int32) + d],
                            v,
                        )
                pltpu.sync_copy(dst0.at[pl.ds(0, tail)],
                                wl_hbm.at[pl.ds(base, tail)])

        def fire_in(i, b):
            bid = i * NW + wid

            @pl.when(bid < n_full)
            def _():
                pltpu.async_copy(wt_hbm.at[:, pl.ds(bid * TB, TB)],
                                 srcs[b], isems[b])

        fire_in(0, 0)
        fire_in(1, 1)

        @pl.loop(0, iters, step=2)
        def blk_group(g):
            for b in range(2):
                i = g + b
                bid = i * NW + wid

                @pl.when(bid < n_full)
                def _():
                    pltpu.make_async_copy(
                        wt_hbm.at[:, pl.ds(bid * TB, TB)], srcs[b], isems[b]
                    ).wait()

                    @pl.when(i >= 2)
                    def _():
                        pltpu.make_async_copy(
                            dsts[b], wl_hbm.at[pl.ds(0, TB)], osems[b]
                        ).wait()

                    src, dst = srcs[b], dsts[b]

                    @plsc.parallel_loop(0, DIM, unroll=2)
                    def d_loop(d):
                        dcol = jnp.full((LANES,), 0, jnp.int32) + d
                        for tg in range(TB // LANES):
                            v = src[d, pl.ds(tg * LANES, LANES)] * SCALE
                            plsc.store_scatter(
                                dst, [iota + tg * LANES, dcol], v)

                    pltpu.async_copy(dst, wl_hbm.at[pl.ds(bid * TB, TB)],
                                     osems[b])
                    fire_in(i + 2, b)

        for b in range(2):
            pltpu.make_async_copy(
                dsts[b], wl_hbm.at[pl.ds(0, TB)], osems[b]
            ).wait()

    return tr_kernel


NBG = 4
NBO = 2


@functools.lru_cache(maxsize=None)
def _build_gather(n_rows: int, n_cols: int, vocab: int):
    rows_per_w = n_rows // NW
    assert rows_per_w * NW == n_rows and n_cols % NBG == 0

    mesh = plsc.VectorSubcoreMesh(core_axis_name="c", subcore_axis_name="s")

    scratch = (
        [pltpu.VMEM((rows_per_w, n_cols), jnp.int32),
         pltpu.VMEM((n_cols, rows_per_w), jnp.int32)]
        + [pltpu.VMEM((rows_per_w, DIM), jnp.float32) for _ in range(NBG)]
        + [pltpu.VMEM((DIM, rows_per_w), jnp.float32) for _ in range(NBO)]
        + [pltpu.SemaphoreType.DMA for _ in range(NBG + NBO)]
    )

    @functools.partial(
        pl.kernel,
        mesh=mesh,
        compiler_params=pltpu.CompilerParams(use_tc_tiling_on_sc=False, needs_layout_passes=False),
        out_type=jax.ShapeDtypeStruct((n_cols, DIM, n_rows), jnp.float32),
        scratch_types=scratch,
    )
    def g_kernel(tok_hbm, wl_hbm, ot_hbm, tok_v, tokT_v, *bufs):
        gbufs = bufs[:NBG]
        obufs = bufs[NBG:NBG + NBO]
        gsems = bufs[NBG + NBO:2 * NBG + NBO]
        osems = bufs[2 * NBG + NBO:]

        wid = lax.axis_index("s") * NC + lax.axis_index("c")
        b0 = wid * rows_per_w
        iota = lax.iota(jnp.int32, LANES)

        pltpu.sync_copy(tok_hbm.at[pl.ds(b0, rows_per_w)], tok_v)

        @plsc.parallel_loop(0, n_cols, unroll=2)
        def s_loop(s):
            scol = jnp.full((LANES,), 0, jnp.int32) + s
            for tg in range(rows_per_w // LANES):
                v = plsc.load_gather(tok_v, [iota + tg * LANES, scol])
                tokT_v[s, pl.ds(tg * LANES, LANES)] = v

        def fire_gather(s, b):
            pltpu.async_copy(wl_hbm.at[tokT_v.at[s]], gbufs[b], gsems[b])

        for b in range(NBG):
            fire_gather(b, b)

        @pl.loop(0, n_cols, step=NBG)
        def s_group(g):
            for b in range(NBG):
                s = g + b
                o = b % NBO
                pltpu.make_async_copy(
                    wl_hbm.at[tokT_v.at[s]], gbufs[b], gsems[b]
                ).wait()

                @pl.when(s >= NBO)
                def _():
                    pltpu.make_async_copy(
                        obufs[o], ot_hbm.at[0, :, pl.ds(b0, rows_per_w)],
                        osems[o],
                    ).wait()

                gbuf, obuf = gbufs[b], obufs[o]

                @plsc.parallel_loop(0, DIM, unroll=2)
                def d_loop(d):
                    dcol = jnp.full((LANES,), 0, jnp.int32) + d
                    for tg in range(rows_per_w // LANES):
                        v = plsc.load_gather(gbuf, [iota + tg * LANES, dcol])
                        obuf[d, pl.ds(tg * LANES, LANES)] = v

                pltpu.async_copy(
                    obuf, ot_hbm.at[s, :, pl.ds(b0, rows_per_w)], osems[o])

                @pl.when(s + NBG < n_cols)
                def _():
                    fire_gather(s + NBG, b)

        for o in range(NBO):
            pltpu.make_async_copy(
                obufs[o], ot_hbm.at[0, :, pl.ds(b0, rows_per_w)], osems[o]
            ).wait()

    return g_kernel


def kernel(tokens, W):
    n_rows, n_cols = tokens.shape
    vocab = W.shape[0]
    wt = jnp.transpose(W)
    wl = _build_transpose(vocab)(wt)
    out_t = _build_gather(n_rows, n_cols, vocab)(tokens.astype(jnp.int32), wl)
    return lax.transpose(out_t, (2, 0, 1))


# speedup vs baseline: 5.4546x; 5.4546x over previous
import functools
import math

import jax
import jax.numpy as jnp
from jax import lax
from jax.experimental import pallas as pl
from jax.experimental.pallas import tpu as pltpu
from jax.experimental.pallas import tpu_sc as plsc

DIM = 64
SCALE = math.sqrt(DIM)

NC = 2
NS = 16
NW = NC * NS
LANES = 16
NB = 4
SPLIT = 128


@functools.lru_cache(maxsize=None)
def _build(n_rows: int, n_cols: int, vocab: int):
    rows_per_w = n_rows // NW
    assert rows_per_w * NW == n_rows and rows_per_w % NB == 0
    rest = n_cols - SPLIT
    assert 0 < rest <= SPLIT and SPLIT % 8 == 0

    mesh = plsc.VectorSubcoreMesh(core_axis_name="c", subcore_axis_name="s")

    scratch = (
        [pltpu.VMEM((rows_per_w, n_cols), jnp.int32)]
        + [pltpu.VMEM((n_cols, DIM), jnp.float32) for _ in range(2 * NB)]
        + [pltpu.SemaphoreType.DMA for _ in range(2 * NB)]
    )

    @functools.partial(
        pl.kernel,
        mesh=mesh,
        compiler_params=pltpu.CompilerParams(use_tc_tiling_on_sc=False),
        out_type=jax.ShapeDtypeStruct((n_rows, n_cols, DIM), jnp.float32),
        scratch_types=scratch,
    )
    def emb_kernel(tok_hbm, table_hbm, out_hbm, tok_v, *bufs):
        rows_in = bufs[:NB]
        rows_out = bufs[NB:2 * NB]
        in_sem = bufs[2 * NB:3 * NB]
        out_sem = bufs[3 * NB:]

        wid = lax.axis_index("s") * NC + lax.axis_index("c")
        wbase = wid * rows_per_w
        pltpu.sync_copy(tok_hbm.at[pl.ds(wbase, rows_per_w)], tok_v)

        def fire_gather(r, b):
            pltpu.async_copy(
                table_hbm.at[tok_v.at[r, pl.ds(0, SPLIT)]],
                rows_in[b].at[pl.ds(0, SPLIT)],
                in_sem[b],
            )
            pltpu.async_copy(
                table_hbm.at[tok_v.at[r, pl.ds(SPLIT, rest)]],
                rows_in[b].at[pl.ds(SPLIT, rest)],
                in_sem[b],
            )

        def wait_gather(r, b):
            pltpu.make_async_copy(
                table_hbm.at[tok_v.at[r, pl.ds(0, SPLIT)]],
                rows_in[b].at[pl.ds(0, SPLIT)],
                in_sem[b],
            ).wait()
            pltpu.make_async_copy(
                table_hbm.at[tok_v.at[r, pl.ds(SPLIT, rest)]],
                rows_in[b].at[pl.ds(SPLIT, rest)],
                in_sem[b],
            ).wait()

        for b in range(NB):
            fire_gather(b, b)

        @pl.loop(0, rows_per_w, step=NB)
        def row_group(g):
            for b in range(NB):
                r = g + b
                wait_gather(r, b)

                @pl.when(r >= NB)
                def _():
                    pltpu.make_async_copy(
                        rows_out[b], out_hbm.at[wbase], out_sem[b]
                    ).wait()

                src = rows_in[b]
                dst = rows_out[b]

                @plsc.parallel_loop(0, n_cols, unroll=8)
                def scale_row(t):
                    for c in range(DIM // LANES):
                        sl = pl.ds(c * LANES, LANES)
                        dst[t, sl] = src[t, sl] * SCALE

                pltpu.async_copy(dst, out_hbm.at[wbase + r], out_sem[b])

                @pl.when(r + NB < rows_per_w)
                def _():
                    fire_gather(r + NB, b)

        for b in range(NB):
            pltpu.make_async_copy(
                rows_out[b], out_hbm.at[wbase], out_sem[b]
            ).wait()

    return emb_kernel


def kernel(tokens, W):
    n_rows, n_cols = tokens.shape
    out = _build(n_rows, n_cols, W.shape[0])(tokens.astype(jnp.int32), W)
    return out
